# baseline (speedup 1.0000x reference)
import functools

import jax
import jax.numpy as jnp
from jax.experimental import pallas as pl
from jax.experimental.pallas import tpu as pltpu

MEMORY_UNIT = 256
MAX_MEMORY = 100000
X_DIM = 256
OUT_DIM = 512
UPDATE_SIZE = 3 + MEMORY_UNIT
JUMP_THRESHOLD = 0.5
MIN_SIM_TO_JUMP = 0.5

ROWS_PER_BLOCK = 2000
NUM_BLOCKS = MAX_MEMORY // ROWS_PER_BLOCK


def _controller_kernel(xj_ref, w_ref, b_ref, out_ref):
    out_ref[...] = (
        jax.lax.dot_general(
            xj_ref[...], w_ref[...], (((1,), (1,)), ((), ())),
            preferred_element_type=jnp.float32,
            precision=jax.lax.Precision.HIGHEST,
        )
        + b_ref[...]
    )


def _scan_kernel(sjw_ref, m_ref, mem_ref, mem2_ref, head_ref,
                 best_ref, pos_ref):
    i = pl.program_id(0)
    blk = mem_ref[...]
    m_row = m_ref[...]
    w = sjw_ref[2]

    rows = jax.lax.broadcasted_iota(jnp.int32, (ROWS_PER_BLOCK, 1), 0)
    overwrite = (rows == 0) & (i == 0) & (w > 0.5)
    blk = jnp.where(overwrite, m_row, blk)
    mem2_ref[...] = blk

    d = blk - m_row
    d2 = jnp.sum(d * d, axis=1, keepdims=True)
    sims = 1.0 - jnp.sqrt(d2) * (1.0 / MEMORY_UNIT)
    local_best = jnp.max(sims)
    local_pos = jnp.min(
        jnp.where(sims == local_best, rows, ROWS_PER_BLOCK)
    ) + i * ROWS_PER_BLOCK

    @pl.when(i == 0)
    def _():
        best_ref[0] = -jnp.inf
        pos_ref[0] = 0

    better = local_best > best_ref[0]
    best_ref[0] = jnp.where(better, local_best, best_ref[0])
    pos_ref[0] = jnp.where(better, local_pos, pos_ref[0])

    @pl.when(i == NUM_BLOCKS - 1)
    def _():
        s = sjw_ref[0]
        j = sjw_ref[1]
        jumped = jnp.where(best_ref[0] > MIN_SIM_TO_JUMP, pos_ref[0], 0)
        head0 = jnp.where(j > JUMP_THRESHOLD, jumped, 0)
        shift = jnp.floor(s * 3.0 - 1e-9).astype(jnp.int32) - 1
        head_ref[0] = jnp.mod(head0 + shift, MAX_MEMORY)


def _gather_kernel(head_ref, mem2_ref, out_ref):
    del head_ref
    out_ref[...] = mem2_ref[0]


def kernel(x, W, b, memory, previous_read, interpret=False):
    xj = jnp.concatenate([x, previous_read[None, :]], axis=1)
    out = pl.pallas_call(
        _controller_kernel,
        out_shape=jax.ShapeDtypeStruct((1, OUT_DIM), jnp.float32),
        interpret=interpret,
    )(xj, W, b[None, :])[0]

    y = out[: OUT_DIM - UPDATE_SIZE]
    sjw = out[OUT_DIM - UPDATE_SIZE : OUT_DIM - UPDATE_SIZE + 3]
    m = out[OUT_DIM - UPDATE_SIZE + 3 :][None, :]

    mem2, head = pl.pallas_call(
        _scan_kernel,
        grid=(NUM_BLOCKS,),
        in_specs=[
            pl.BlockSpec(memory_space=pltpu.SMEM),
            pl.BlockSpec((1, MEMORY_UNIT), lambda i: (0, 0)),
            pl.BlockSpec((ROWS_PER_BLOCK, MEMORY_UNIT), lambda i: (i, 0)),
        ],
        out_specs=[
            pl.BlockSpec((ROWS_PER_BLOCK, MEMORY_UNIT), lambda i: (i, 0)),
            pl.BlockSpec(memory_space=pltpu.SMEM),
        ],
        out_shape=[
            jax.ShapeDtypeStruct((MAX_MEMORY, MEMORY_UNIT), jnp.float32),
            jax.ShapeDtypeStruct((1,), jnp.int32),
        ],
        scratch_shapes=[
            pltpu.SMEM((1,), jnp.float32),
            pltpu.SMEM((1,), jnp.int32),
        ],
        compiler_params=pltpu.CompilerParams(
            dimension_semantics=("arbitrary",),
        ),
        interpret=interpret,
    )(sjw, m, memory)

    read = pl.pallas_call(
        _gather_kernel,
        grid_spec=pltpu.PrefetchScalarGridSpec(
            num_scalar_prefetch=1,
            grid=(1,),
            in_specs=[
                pl.BlockSpec((1, 1, MEMORY_UNIT), lambda i, h: (h[0], 0, 0))
            ],
            out_specs=pl.BlockSpec((1, MEMORY_UNIT), lambda i, h: (0, 0)),
        ),
        out_shape=jax.ShapeDtypeStruct((1, MEMORY_UNIT), jnp.float32),
        interpret=interpret,
    )(head, mem2.reshape(MAX_MEMORY, 1, MEMORY_UNIT))[0]

    return y, read, mem2


# speedup vs baseline: 3.8516x; 3.8516x over previous
import functools

import jax
import jax.numpy as jnp
from jax.experimental import pallas as pl
from jax.experimental.pallas import tpu as pltpu

MEMORY_UNIT = 256
MAX_MEMORY = 100000
X_DIM = 256
OUT_DIM = 512
UPDATE_SIZE = 3 + MEMORY_UNIT
JUMP_THRESHOLD = 0.5
MIN_SIM_TO_JUMP = 0.5

ROWS_PER_BLOCK = 2000
NUM_BLOCKS = MAX_MEMORY // ROWS_PER_BLOCK


def _controller_kernel(xj_ref, w_ref, b_ref, out_ref):
    out_ref[...] = (
        jax.lax.dot_general(
            xj_ref[...], w_ref[...], (((1,), (1,)), ((), ())),
            preferred_element_type=jnp.float32,
            precision=jax.lax.Precision.HIGHEST,
        )
        + b_ref[...]
    )


def _scan_kernel(sjw_ref, m_ref, mem_ref, mem2_ref, head_ref,
                 best_ref, pos_ref):
    i = pl.program_id(0)
    blk = mem_ref[...]
    m_row = m_ref[...]
    w = sjw_ref[2]

    rows = jax.lax.broadcasted_iota(jnp.int32, (ROWS_PER_BLOCK, 1), 0)
    overwrite = (rows == 0) & (i == 0) & (w > 0.5)
    blk = jnp.where(overwrite, m_row, blk)
    mem2_ref[...] = blk

    d = blk - m_row
    d2 = jnp.sum(d * d, axis=1, keepdims=True)
    sims = 1.0 - jnp.sqrt(d2) * (1.0 / MEMORY_UNIT)
    local_best = jnp.max(sims)
    local_pos = jnp.min(
        jnp.where(sims == local_best, rows, ROWS_PER_BLOCK)
    ) + i * ROWS_PER_BLOCK

    @pl.when(i == 0)
    def _():
        best_ref[0] = -jnp.inf
        pos_ref[0] = 0

    better = local_best > best_ref[0]
    best_ref[0] = jnp.where(better, local_best, best_ref[0])
    pos_ref[0] = jnp.where(better, local_pos, pos_ref[0])

    @pl.when(i == NUM_BLOCKS - 1)
    def _():
        s = sjw_ref[0]
        j = sjw_ref[1]
        jumped = jnp.where(best_ref[0] > MIN_SIM_TO_JUMP, pos_ref[0], 0)
        head0 = jnp.where(j > JUMP_THRESHOLD, jumped, 0)
        shift = jnp.floor(s * 3.0 - 1e-9).astype(jnp.int32) - 1
        head = jnp.mod(head0 + shift, MAX_MEMORY)
        head_ref[0] = head // 8
        head_ref[1] = head % 8


def _gather_kernel(head_ref, mem2_ref, out_ref):
    out_ref[...] = mem2_ref[pl.ds(head_ref[1], 1), :]


def kernel(x, W, b, memory, previous_read, interpret=False):
    xj = jnp.concatenate([x, previous_read[None, :]], axis=1)
    out = pl.pallas_call(
        _controller_kernel,
        out_shape=jax.ShapeDtypeStruct((1, OUT_DIM), jnp.float32),
        interpret=interpret,
    )(xj, W, b[None, :])[0]

    y = out[: OUT_DIM - UPDATE_SIZE]
    sjw = out[OUT_DIM - UPDATE_SIZE : OUT_DIM - UPDATE_SIZE + 3]
    m = out[OUT_DIM - UPDATE_SIZE + 3 :][None, :]

    mem2, head = pl.pallas_call(
        _scan_kernel,
        grid=(NUM_BLOCKS,),
        in_specs=[
            pl.BlockSpec(memory_space=pltpu.SMEM),
            pl.BlockSpec((1, MEMORY_UNIT), lambda i: (0, 0)),
            pl.BlockSpec((ROWS_PER_BLOCK, MEMORY_UNIT), lambda i: (i, 0)),
        ],
        out_specs=[
            pl.BlockSpec((ROWS_PER_BLOCK, MEMORY_UNIT), lambda i: (i, 0)),
            pl.BlockSpec(memory_space=pltpu.SMEM),
        ],
        out_shape=[
            jax.ShapeDtypeStruct((MAX_MEMORY, MEMORY_UNIT), jnp.float32),
            jax.ShapeDtypeStruct((2,), jnp.int32),
        ],
        scratch_shapes=[
            pltpu.SMEM((1,), jnp.float32),
            pltpu.SMEM((1,), jnp.int32),
        ],
        compiler_params=pltpu.CompilerParams(
            dimension_semantics=("arbitrary",),
        ),
        interpret=interpret,
    )(sjw, m, memory)

    read = pl.pallas_call(
        _gather_kernel,
        grid_spec=pltpu.PrefetchScalarGridSpec(
            num_scalar_prefetch=1,
            grid=(1,),
            in_specs=[
                pl.BlockSpec((8, MEMORY_UNIT), lambda i, h: (h[0], 0))
            ],
            out_specs=pl.BlockSpec((1, MEMORY_UNIT), lambda i, h: (0, 0)),
        ),
        out_shape=jax.ShapeDtypeStruct((1, MEMORY_UNIT), jnp.float32),
        interpret=interpret,
    )(head, mem2)[0]

    return y, read, mem2


# speedup vs baseline: 4.2948x; 1.1151x over previous
import jax
import jax.numpy as jnp
from jax.experimental import pallas as pl
from jax.experimental.pallas import tpu as pltpu

MEMORY_UNIT = 256
MAX_MEMORY = 100000
OUT_DIM = 512
UPDATE_SIZE = 3 + MEMORY_UNIT
Y_DIM = OUT_DIM - UPDATE_SIZE
JUMP_THRESHOLD = 0.5
MIN_SIM_TO_JUMP = 0.5

ROWS_PER_BLOCK = 4000
NUM_BLOCKS = MAX_MEMORY // ROWS_PER_BLOCK


def _ntm_kernel(xj_ref, w_mat_ref, b_ref, mem_ref, mem_any_ref,
                y_ref, read_ref, mem2_ref,
                ctrl_ref, land_ref, best_ref, pos_ref, sem):
    i = pl.program_id(0)

    @pl.when(i == 0)
    def _():
        out = jax.lax.dot_general(
            xj_ref[...], w_mat_ref[...], (((1,), (1,)), ((), ())),
            preferred_element_type=jnp.float32,
            precision=jax.lax.Precision.HIGHEST,
        ) + b_ref[...]
        ctrl_ref[...] = out
        y_ref[...] = out[:, :Y_DIM]
        best_ref[0] = -jnp.inf
        pos_ref[0] = 0

    w = ctrl_ref[0, Y_DIM + 2]
    m_row = ctrl_ref[0:1, Y_DIM + 3:]

    blk = mem_ref[...]
    mem2_ref[...] = blk

    @pl.when((i == 0) & (w > 0.5))
    def _():
        mem2_ref[0:1, :] = m_row

    d = blk - m_row
    d2 = jnp.sum(d * d, axis=1, keepdims=True)
    sims = 1.0 - jnp.sqrt(d2) * (1.0 / MEMORY_UNIT)
    local_best = jnp.max(sims)
    rows = jax.lax.broadcasted_iota(jnp.int32, (ROWS_PER_BLOCK, 1), 0)
    local_pos = jnp.min(
        jnp.where(sims == local_best, rows, ROWS_PER_BLOCK)
    ) + i * ROWS_PER_BLOCK

    better = local_best > best_ref[0]
    best_ref[0] = jnp.where(better, local_best, best_ref[0])
    pos_ref[0] = jnp.where(better, local_pos, pos_ref[0])

    @pl.when(i == NUM_BLOCKS - 1)
    def _():
        s = ctrl_ref[0, Y_DIM]
        j = ctrl_ref[0, Y_DIM + 1]
        jumped = jnp.where(
            w > 0.5, 0,
            jnp.where(best_ref[0] > MIN_SIM_TO_JUMP, pos_ref[0], 0),
        )
        head0 = jnp.where(j > JUMP_THRESHOLD, jumped, 0)
        shift = jnp.floor(s * 3.0 - 1e-9).astype(jnp.int32) - 1
        head = jnp.mod(head0 + shift, MAX_MEMORY)
        copy = pltpu.make_async_copy(
            mem_any_ref.at[pl.ds(head, 1)], land_ref, sem)
        copy.start()
        copy.wait()
        read_ref[...] = jnp.where(
            (head == 0) & (w > 0.5), m_row, land_ref[...])


def kernel(x, W, b, memory, previous_read, interpret=False):
    xj = jnp.concatenate([x, previous_read[None, :]], axis=1)

    y, read, mem2 = pl.pallas_call(
        _ntm_kernel,
        grid=(NUM_BLOCKS,),
        in_specs=[
            pl.BlockSpec((1, OUT_DIM), lambda i: (0, 0)),
            pl.BlockSpec((OUT_DIM, OUT_DIM), lambda i: (0, 0)),
            pl.BlockSpec((1, OUT_DIM), lambda i: (0, 0)),
            pl.BlockSpec((ROWS_PER_BLOCK, MEMORY_UNIT), lambda i: (i, 0)),
            pl.BlockSpec(memory_space=pl.ANY),
        ],
        out_specs=[
            pl.BlockSpec((1, Y_DIM), lambda i: (0, 0)),
            pl.BlockSpec((1, MEMORY_UNIT), lambda i: (0, 0)),
            pl.BlockSpec((ROWS_PER_BLOCK, MEMORY_UNIT), lambda i: (i, 0)),
        ],
        out_shape=[
            jax.ShapeDtypeStruct((1, Y_DIM), jnp.float32),
            jax.ShapeDtypeStruct((1, MEMORY_UNIT), jnp.float32),
            jax.ShapeDtypeStruct((MAX_MEMORY, MEMORY_UNIT), jnp.float32),
        ],
        scratch_shapes=[
            pltpu.VMEM((1, OUT_DIM), jnp.float32),
            pltpu.VMEM((1, MEMORY_UNIT), jnp.float32),
            pltpu.SMEM((1,), jnp.float32),
            pltpu.SMEM((1,), jnp.int32),
            pltpu.SemaphoreType.DMA,
        ],
        compiler_params=pltpu.CompilerParams(
            dimension_semantics=("arbitrary",),
        ),
        interpret=interpret,
    )(xj, W, b[None, :], memory, memory)

    return y[0], read[0], mem2


# speedup vs baseline: 4.4395x; 1.0337x over previous
import jax
import jax.numpy as jnp
from jax.experimental import pallas as pl
from jax.experimental.pallas import tpu as pltpu

MEMORY_UNIT = 256
MAX_MEMORY = 100000
OUT_DIM = 512
UPDATE_SIZE = 3 + MEMORY_UNIT
Y_DIM = OUT_DIM - UPDATE_SIZE
JUMP_THRESHOLD = 0.5
MIN_SIM_TO_JUMP = 0.5

ROWS_PER_BLOCK = 5000
NUM_BLOCKS = MAX_MEMORY // ROWS_PER_BLOCK


def _ntm_kernel(xj_ref, w_mat_ref, b_ref, mem_ref, mem_any_ref,
                y_ref, read_ref, mem2_ref,
                ctrl_ref, land_ref, best_ref, pos_ref, sem):
    i = pl.program_id(0)

    @pl.when(i == 0)
    def _():
        out = jax.lax.dot_general(
            xj_ref[...], w_mat_ref[...], (((1,), (1,)), ((), ())),
            preferred_element_type=jnp.float32,
            precision=jax.lax.Precision.HIGHEST,
        ) + b_ref[...]
        ctrl_ref[...] = out
        y_ref[...] = out[:, :Y_DIM]
        best_ref[0] = -jnp.inf
        pos_ref[0] = 0

    w = ctrl_ref[0, Y_DIM + 2]
    m_row = ctrl_ref[0:1, Y_DIM + 3:]

    blk = mem_ref[...]
    mem2_ref[...] = blk

    @pl.when((i == 0) & (w > 0.5))
    def _():
        mem2_ref[0:1, :] = m_row

    d = blk - m_row
    d2 = jnp.sum(d * d, axis=1, keepdims=True)
    sims = 1.0 - jnp.sqrt(d2) * (1.0 / MEMORY_UNIT)
    local_best = jnp.max(sims)
    rows = jax.lax.broadcasted_iota(jnp.int32, (ROWS_PER_BLOCK, 1), 0)
    local_pos = jnp.min(
        jnp.where(sims == local_best, rows, ROWS_PER_BLOCK)
    ) + i * ROWS_PER_BLOCK

    better = local_best > best_ref[0]
    best_ref[0] = jnp.where(better, local_best, best_ref[0])
    pos_ref[0] = jnp.where(better, local_pos, pos_ref[0])

    @pl.when(i == NUM_BLOCKS - 1)
    def _():
        s = ctrl_ref[0, Y_DIM]
        j = ctrl_ref[0, Y_DIM + 1]
        jumped = jnp.where(
            w > 0.5, 0,
            jnp.where(best_ref[0] > MIN_SIM_TO_JUMP, pos_ref[0], 0),
        )
        head0 = jnp.where(j > JUMP_THRESHOLD, jumped, 0)
        shift = jnp.floor(s * 3.0 - 1e-9).astype(jnp.int32) - 1
        head = jnp.mod(head0 + shift, MAX_MEMORY)
        copy = pltpu.make_async_copy(
            mem_any_ref.at[pl.ds(head, 1)], land_ref, sem)
        copy.start()
        copy.wait()
        read_ref[...] = jnp.where(
            (head == 0) & (w > 0.5), m_row, land_ref[...])


def kernel(x, W, b, memory, previous_read, interpret=False):
    xj = jnp.concatenate([x, previous_read[None, :]], axis=1)

    y, read, mem2 = pl.pallas_call(
        _ntm_kernel,
        grid=(NUM_BLOCKS,),
        in_specs=[
            pl.BlockSpec((1, OUT_DIM), lambda i: (0, 0)),
            pl.BlockSpec((OUT_DIM, OUT_DIM), lambda i: (0, 0)),
            pl.BlockSpec((1, OUT_DIM), lambda i: (0, 0)),
            pl.BlockSpec((ROWS_PER_BLOCK, MEMORY_UNIT), lambda i: (i, 0)),
            pl.BlockSpec(memory_space=pl.ANY),
        ],
        out_specs=[
            pl.BlockSpec((1, Y_DIM), lambda i: (0, 0)),
            pl.BlockSpec((1, MEMORY_UNIT), lambda i: (0, 0)),
            pl.BlockSpec((ROWS_PER_BLOCK, MEMORY_UNIT), lambda i: (i, 0)),
        ],
        out_shape=[
            jax.ShapeDtypeStruct((1, Y_DIM), jnp.float32),
            jax.ShapeDtypeStruct((1, MEMORY_UNIT), jnp.float32),
            jax.ShapeDtypeStruct((MAX_MEMORY, MEMORY_UNIT), jnp.float32),
        ],
        scratch_shapes=[
            pltpu.VMEM((1, OUT_DIM), jnp.float32),
            pltpu.VMEM((1, MEMORY_UNIT), jnp.float32),
            pltpu.SMEM((1,), jnp.float32),
            pltpu.SMEM((1,), jnp.int32),
            pltpu.SemaphoreType.DMA,
        ],
        compiler_params=pltpu.CompilerParams(
            dimension_semantics=("arbitrary",),
        ),
        interpret=interpret,
    )(xj, W, b[None, :], memory, memory)

    return y[0], read[0], mem2


# speedup vs baseline: 4.6661x; 1.0510x over previous
import jax
import jax.numpy as jnp
from jax.experimental import pallas as pl
from jax.experimental.pallas import tpu as pltpu

MEMORY_UNIT = 256
MAX_MEMORY = 100000
OUT_DIM = 512
UPDATE_SIZE = 3 + MEMORY_UNIT
Y_DIM = OUT_DIM - UPDATE_SIZE
JUMP_THRESHOLD = 0.5
MIN_SIM_TO_JUMP = 0.5

ROWS_PER_BLOCK = 10000
NUM_BLOCKS = MAX_MEMORY // ROWS_PER_BLOCK


def _ntm_kernel(xj_ref, w_mat_ref, b_ref, mem_ref, mem_any_ref,
                y_ref, read_ref, mem2_ref,
                ctrl_ref, land_ref, best_ref, pos_ref, sem):
    i = pl.program_id(0)

    @pl.when(i == 0)
    def _():
        out = jax.lax.dot_general(
            xj_ref[...], w_mat_ref[...], (((1,), (1,)), ((), ())),
            preferred_element_type=jnp.float32,
            precision=jax.lax.Precision.HIGHEST,
        ) + b_ref[...]
        ctrl_ref[...] = out
        y_ref[...] = out[:, :Y_DIM]
        best_ref[0] = -jnp.inf
        pos_ref[0] = 0

    w = ctrl_ref[0, Y_DIM + 2]
    m_row = ctrl_ref[0:1, Y_DIM + 3:]

    blk = mem_ref[...]
    mem2_ref[...] = blk

    @pl.when((i == 0) & (w > 0.5))
    def _():
        mem2_ref[0:1, :] = m_row

    d = blk - m_row
    d2 = jnp.sum(d * d, axis=1, keepdims=True)
    sims = 1.0 - jnp.sqrt(d2) * (1.0 / MEMORY_UNIT)
    local_best = jnp.max(sims)
    rows = jax.lax.broadcasted_iota(jnp.int32, (ROWS_PER_BLOCK, 1), 0)
    local_pos = jnp.min(
        jnp.where(sims == local_best, rows, ROWS_PER_BLOCK)
    ) + i * ROWS_PER_BLOCK

    better = local_best > best_ref[0]
    best_ref[0] = jnp.where(better, local_best, best_ref[0])
    pos_ref[0] = jnp.where(better, local_pos, pos_ref[0])

    @pl.when(i == NUM_BLOCKS - 1)
    def _():
        s = ctrl_ref[0, Y_DIM]
        j = ctrl_ref[0, Y_DIM + 1]
        jumped = jnp.where(
            w > 0.5, 0,
            jnp.where(best_ref[0] > MIN_SIM_TO_JUMP, pos_ref[0], 0),
        )
        head0 = jnp.where(j > JUMP_THRESHOLD, jumped, 0)
        shift = jnp.floor(s * 3.0 - 1e-9).astype(jnp.int32) - 1
        head = jnp.mod(head0 + shift, MAX_MEMORY)
        copy = pltpu.make_async_copy(
            mem_any_ref.at[pl.ds(head, 1)], land_ref, sem)
        copy.start()
        copy.wait()
        read_ref[...] = jnp.where(
            (head == 0) & (w > 0.5), m_row, land_ref[...])


def kernel(x, W, b, memory, previous_read, interpret=False):
    xj = jnp.concatenate([x, previous_read[None, :]], axis=1)

    y, read, mem2 = pl.pallas_call(
        _ntm_kernel,
        grid=(NUM_BLOCKS,),
        in_specs=[
            pl.BlockSpec((1, OUT_DIM), lambda i: (0, 0)),
            pl.BlockSpec((OUT_DIM, OUT_DIM), lambda i: (0, 0)),
            pl.BlockSpec((1, OUT_DIM), lambda i: (0, 0)),
            pl.BlockSpec((ROWS_PER_BLOCK, MEMORY_UNIT), lambda i: (i, 0)),
            pl.BlockSpec(memory_space=pl.ANY),
        ],
        out_specs=[
            pl.BlockSpec((1, Y_DIM), lambda i: (0, 0)),
            pl.BlockSpec((1, MEMORY_UNIT), lambda i: (0, 0)),
            pl.BlockSpec((ROWS_PER_BLOCK, MEMORY_UNIT), lambda i: (i, 0)),
        ],
        out_shape=[
            jax.ShapeDtypeStruct((1, Y_DIM), jnp.float32),
            jax.ShapeDtypeStruct((1, MEMORY_UNIT), jnp.float32),
            jax.ShapeDtypeStruct((MAX_MEMORY, MEMORY_UNIT), jnp.float32),
        ],
        scratch_shapes=[
            pltpu.VMEM((1, OUT_DIM), jnp.float32),
            pltpu.VMEM((1, MEMORY_UNIT), jnp.float32),
            pltpu.SMEM((1,), jnp.float32),
            pltpu.SMEM((1,), jnp.int32),
            pltpu.SemaphoreType.DMA,
        ],
        compiler_params=pltpu.CompilerParams(
            dimension_semantics=("arbitrary",),
        ),
        interpret=interpret,
    )(xj, W, b[None, :], memory, memory)

    return y[0], read[0], mem2
